# Initial kernel scaffold; baseline (speedup 1.0000x reference)
import jax
import jax.numpy as jnp
from jax.experimental import pallas as pl


def kernel(sample, W_enc, speaker_vectors):
    raise NotImplementedError("write your pallas kernel here")


# speedup vs baseline: 8.3819x; 8.3819x over previous
import functools

import jax
import jax.numpy as jnp
from jax.experimental import pallas as pl
from jax.experimental.pallas import tpu as pltpu

_Q, _D_IN, _D, _S, _V, _TOPK = 1024, 512, 256, 500, 64, 4
_S_PAD = 512
_QT = 256
_BIG = 3.0e38


def _conf_kernel(sample_ref, w_ref, keys_ref,
                 mean_ref, topk_ref, min_ref, ml_ref, tl_ref, nl_ref,
                 vec_s, q2_s, sum_s, m1_s, m2_s, m3_s, m4_s):
    v = pl.program_id(1)

    @pl.when(v == 0)
    def _init():
        vec = jnp.tanh(jnp.dot(sample_ref[...], w_ref[...],
                               preferred_element_type=jnp.float32))
        vec_s[...] = vec
        q2 = jnp.sum(vec * vec, axis=1, keepdims=True)
        q2_s[...] = jnp.broadcast_to(q2, q2_s.shape)
        sum_s[...] = jnp.zeros_like(sum_s)
        big = jnp.full(m1_s.shape, _BIG, jnp.float32)
        m1_s[...] = big
        m2_s[...] = big
        m3_s[...] = big
        m4_s[...] = big

    keys = keys_ref[0]
    k2 = jnp.sum(keys * keys, axis=0, keepdims=True)
    vec = vec_s[...]
    dot = jnp.dot(vec, keys, preferred_element_type=jnp.float32)
    dist = q2_s[:, 0:1] + k2 - 2.0 * dot
    lane = jax.lax.broadcasted_iota(jnp.int32, (1, _S_PAD), 1)
    dist = jnp.where(lane >= _S, _BIG, dist)

    sum_s[...] = sum_s[...] + dist

    x = dist
    m1 = m1_s[...]
    m1_s[...] = jnp.minimum(m1, x)
    x = jnp.maximum(m1, x)
    m2 = m2_s[...]
    m2_s[...] = jnp.minimum(m2, x)
    x = jnp.maximum(m2, x)
    m3 = m3_s[...]
    m3_s[...] = jnp.minimum(m3, x)
    x = jnp.maximum(m3, x)
    m4 = m4_s[...]
    m4_s[...] = jnp.minimum(m4, x)

    @pl.when(v == _V - 1)
    def _finalize():
        mean = sum_s[...] * (1.0 / _V)
        m1v = m1_s[...]
        topk = (((m1v + m2_s[...]) + m3_s[...]) + m4_s[...]) * (1.0 / _TOPK)
        mean_ref[...] = mean
        topk_ref[...] = topk
        min_ref[...] = m1v
        ml = jnp.argmin(mean, axis=1).astype(jnp.int32)
        tl = jnp.argmin(topk, axis=1).astype(jnp.int32)
        nl = jnp.argmin(m1v, axis=1).astype(jnp.int32)
        ml_ref[...] = jnp.broadcast_to(ml[:, None], ml_ref.shape)
        tl_ref[...] = jnp.broadcast_to(tl[:, None], tl_ref.shape)
        nl_ref[...] = jnp.broadcast_to(nl[:, None], nl_ref.shape)


@jax.jit
def kernel(sample, W_enc, speaker_vectors):
    keys_vds = jnp.pad(jnp.transpose(speaker_vectors, (1, 2, 0)),
                       ((0, 0), (0, 0), (0, _S_PAD - _S)))
    grid = (_Q // _QT, _V)
    f32 = jnp.float32
    out = pl.pallas_call(
        _conf_kernel,
        grid=grid,
        in_specs=[
            pl.BlockSpec((_QT, _D_IN), lambda qi, v: (qi, 0)),
            pl.BlockSpec((_D_IN, _D), lambda qi, v: (0, 0)),
            pl.BlockSpec((1, _D, _S_PAD), lambda qi, v: (v, 0, 0)),
        ],
        out_specs=[
            pl.BlockSpec((_QT, _S_PAD), lambda qi, v: (qi, 0)),
            pl.BlockSpec((_QT, _S_PAD), lambda qi, v: (qi, 0)),
            pl.BlockSpec((_QT, _S_PAD), lambda qi, v: (qi, 0)),
            pl.BlockSpec((_QT, 128), lambda qi, v: (qi, 0)),
            pl.BlockSpec((_QT, 128), lambda qi, v: (qi, 0)),
            pl.BlockSpec((_QT, 128), lambda qi, v: (qi, 0)),
        ],
        out_shape=[
            jax.ShapeDtypeStruct((_Q, _S_PAD), f32),
            jax.ShapeDtypeStruct((_Q, _S_PAD), f32),
            jax.ShapeDtypeStruct((_Q, _S_PAD), f32),
            jax.ShapeDtypeStruct((_Q, 128), jnp.int32),
            jax.ShapeDtypeStruct((_Q, 128), jnp.int32),
            jax.ShapeDtypeStruct((_Q, 128), jnp.int32),
        ],
        scratch_shapes=[
            pltpu.VMEM((_QT, _D), f32),
            pltpu.VMEM((_QT, 128), f32),
            pltpu.VMEM((_QT, _S_PAD), f32),
            pltpu.VMEM((_QT, _S_PAD), f32),
            pltpu.VMEM((_QT, _S_PAD), f32),
            pltpu.VMEM((_QT, _S_PAD), f32),
            pltpu.VMEM((_QT, _S_PAD), f32),
        ],
        compiler_params=pltpu.CompilerParams(
            dimension_semantics=("parallel", "arbitrary")),
    )(sample, W_enc, keys_vds)
    mean_o, topk_o, min_o, ml_o, tl_o, nl_o = out
    stats = jnp.stack([mean_o[:, :_S], topk_o[:, :_S], min_o[:, :_S]], axis=-1)
    return stats, ml_o[:, 0], tl_o[:, 0], nl_o[:, 0]


# speedup vs baseline: 10.7770x; 1.2857x over previous
import jax
import jax.numpy as jnp
from jax.experimental import pallas as pl
from jax.experimental.pallas import tpu as pltpu

_Q, _D_IN, _D, _S, _V, _TOPK = 1024, 512, 256, 500, 64, 4
_BIG = 3.0e38
_TDIMS = (((1,), (1,)), ((), ()))


def _conf_kernel(sample_ref, w_ref, keys_ref, k2_ref,
                 mean_ref, topk_ref, min_ref, ml_ref, tl_ref, nl_ref,
                 vecm2_s, q2_s, m1_s, m2_s, m3_s, m4_s,
                 s0, s1, s2, s3, s4, s5, s6, s7):
    v = pl.program_id(0)
    sums = (s0, s1, s2, s3, s4, s5, s6, s7)

    @pl.when(v == 0)
    def _init():
        enc = jnp.tanh(jnp.dot(sample_ref[...], w_ref[...],
                               preferred_element_type=jnp.float32))
        e2 = enc * enc
        acc = e2[:, 0:8]
        for j in range(1, 32):
            acc = acc + e2[:, 8 * j:8 * j + 8]
        t = acc[:, 0:4] + acc[:, 4:8]
        t = t[:, 0:2] + t[:, 2:4]
        q2 = t[:, 0:1] + t[:, 1:2]
        q2_s[...] = jnp.broadcast_to(q2, q2_s.shape)
        vecm2_s[...] = -2.0 * enc
        zero = jnp.zeros(s0.shape, jnp.float32)
        for s in sums:
            s[...] = zero
        big = jnp.full(m1_s.shape, _BIG, jnp.float32)
        m1_s[...] = big
        m2_s[...] = big
        m3_s[...] = big
        m4_s[...] = big

    keys = keys_ref[:, 0, 0, :]
    dotm2 = jax.lax.dot_general(vecm2_s[...], keys, _TDIMS,
                                preferred_element_type=jnp.float32)
    dist = (q2_s[:, 0:1] + k2_ref[0]) + dotm2

    r = jax.lax.rem(v, 8)
    for i, s in enumerate(sums):
        @pl.when(r == i)
        def _acc(s=s):
            s[...] = s[...] + dist

    x = dist
    m1 = m1_s[...]
    m1_s[...] = jnp.minimum(m1, x)
    x = jnp.maximum(m1, x)
    m2 = m2_s[...]
    m2_s[...] = jnp.minimum(m2, x)
    x = jnp.maximum(m2, x)
    m3 = m3_s[...]
    m3_s[...] = jnp.minimum(m3, x)
    x = jnp.maximum(m3, x)
    m4_s[...] = jnp.minimum(m4_s[...], x)

    @pl.when(v == _V - 1)
    def _finalize():
        b0 = s0[...] + s4[...]
        b1 = s1[...] + s5[...]
        b2 = s2[...] + s6[...]
        b3 = s3[...] + s7[...]
        total = (b0 + b2) + (b1 + b3)
        mean = total * (1.0 / _V)
        m1v = m1_s[...]
        topk = ((m1v + m3_s[...]) + (m2_s[...] + m4_s[...])) * (1.0 / _TOPK)
        mean_ref[...] = mean
        topk_ref[...] = topk
        min_ref[...] = m1v
        ml = jnp.argmin(mean, axis=1).astype(jnp.int32)
        tl = jnp.argmin(topk, axis=1).astype(jnp.int32)
        nl = jnp.argmin(m1v, axis=1).astype(jnp.int32)
        ml_ref[...] = jnp.broadcast_to(ml[:, None], ml_ref.shape)
        tl_ref[...] = jnp.broadcast_to(tl[:, None], tl_ref.shape)
        nl_ref[...] = jnp.broadcast_to(nl[:, None], nl_ref.shape)


@jax.jit
def kernel(sample, W_enc, speaker_vectors):
    keys4d = jnp.reshape(speaker_vectors, (_S, _V, 1, _D))
    k2 = jnp.sum(speaker_vectors * speaker_vectors, axis=2)
    k2v = jnp.transpose(k2)[:, None, :]
    f32 = jnp.float32
    out = pl.pallas_call(
        _conf_kernel,
        grid=(_V,),
        in_specs=[
            pl.BlockSpec((_Q, _D_IN), lambda v: (0, 0)),
            pl.BlockSpec((_D_IN, _D), lambda v: (0, 0)),
            pl.BlockSpec((_S, 1, 1, _D), lambda v: (0, v, 0, 0)),
            pl.BlockSpec((1, 1, _S), lambda v: (v, 0, 0)),
        ],
        out_specs=[
            pl.BlockSpec((_Q, _S), lambda v: (0, 0)),
            pl.BlockSpec((_Q, _S), lambda v: (0, 0)),
            pl.BlockSpec((_Q, _S), lambda v: (0, 0)),
            pl.BlockSpec((_Q, 128), lambda v: (0, 0)),
            pl.BlockSpec((_Q, 128), lambda v: (0, 0)),
            pl.BlockSpec((_Q, 128), lambda v: (0, 0)),
        ],
        out_shape=[
            jax.ShapeDtypeStruct((_Q, _S), f32),
            jax.ShapeDtypeStruct((_Q, _S), f32),
            jax.ShapeDtypeStruct((_Q, _S), f32),
            jax.ShapeDtypeStruct((_Q, 128), jnp.int32),
            jax.ShapeDtypeStruct((_Q, 128), jnp.int32),
            jax.ShapeDtypeStruct((_Q, 128), jnp.int32),
        ],
        scratch_shapes=[
            pltpu.VMEM((_Q, _D), f32),
            pltpu.VMEM((_Q, 128), f32),
            pltpu.VMEM((_Q, _S), f32),
            pltpu.VMEM((_Q, _S), f32),
            pltpu.VMEM((_Q, _S), f32),
            pltpu.VMEM((_Q, _S), f32),
        ] + [pltpu.VMEM((_Q, _S), f32) for _ in range(8)],
        compiler_params=pltpu.CompilerParams(
            dimension_semantics=("arbitrary",)),
    )(sample, W_enc, keys4d, k2v)
    mean_o, topk_o, min_o, ml_o, tl_o, nl_o = out
    stats = jnp.stack([mean_o, topk_o, min_o], axis=-1)
    return stats, ml_o[:, 0], tl_o[:, 0], nl_o[:, 0]
